# baseline (speedup 1.0000x reference)
import jax
import jax.numpy as jnp
from jax import lax
from jax.experimental import pallas as pl
from jax.experimental.pallas import tpu as pltpu

N = 8192
D = 64
BM = 512

_DN_T = (((1,), (1,)), ((), ()))


def _outer(xs_hbm, adj_hbm, x_hbm, w_hbm, b_hbm, o_hbm,
           x_vmem, xbf_vmem, w_vmem, b_vmem, sems):
    cp1 = pltpu.make_async_copy(x_hbm, x_vmem, sems.at[0])
    cp2 = pltpu.make_async_copy(w_hbm, w_vmem, sems.at[1])
    cp3 = pltpu.make_async_copy(b_hbm, b_vmem, sems.at[2])
    cp1.start(); cp2.start(); cp3.start()
    cp1.wait(); cp2.wait(); cp3.wait()
    xbf_vmem[...] = x_vmem[...].astype(jnp.bfloat16)

    def inner(xs_ref, adj_ref, o_ref):
        neigh = jnp.dot(adj_ref[...].astype(jnp.bfloat16), xbf_vmem[...],
                        preferred_element_type=jnp.float32)
        acc = lax.dot_general(xs_ref[...], w_vmem[:, :D], _DN_T,
                              preferred_element_type=jnp.float32)
        acc = acc + lax.dot_general(neigh, w_vmem[:, D:], _DN_T,
                                    preferred_element_type=jnp.float32)
        o_ref[...] = jnp.maximum(acc + b_vmem[...], 0.0)

    pipe = pltpu.emit_pipeline(
        inner,
        grid=(N // BM,),
        in_specs=[
            pl.BlockSpec((BM, D), lambda i: (i, 0)),
            pl.BlockSpec((BM, N), lambda i: (i, 0),
                         pipeline_mode=pl.Buffered(buffer_count=3)),
        ],
        out_specs=[pl.BlockSpec((BM, D), lambda i: (i, 0))],
    )
    pipe(xs_hbm, adj_hbm, o_hbm)


@jax.jit
def kernel(x, adj_matrix, W, b):
    b2 = b.reshape(1, D)
    out = pl.pallas_call(
        _outer,
        in_specs=[
            pl.BlockSpec(memory_space=pltpu.HBM),
            pl.BlockSpec(memory_space=pltpu.HBM),
            pl.BlockSpec(memory_space=pltpu.HBM),
            pl.BlockSpec(memory_space=pltpu.HBM),
            pl.BlockSpec(memory_space=pltpu.HBM),
        ],
        out_specs=pl.BlockSpec(memory_space=pltpu.HBM),
        out_shape=jax.ShapeDtypeStruct((N, D), jnp.float32),
        scratch_shapes=[
            pltpu.VMEM((N, D), jnp.float32),
            pltpu.VMEM((N, D), jnp.bfloat16),
            pltpu.VMEM((D, 2 * D), jnp.float32),
            pltpu.VMEM((1, D), jnp.float32),
            pltpu.SemaphoreType.DMA((3,)),
        ],
        compiler_params=pltpu.CompilerParams(
            vmem_limit_bytes=60 * 1024 * 1024,
        ),
    )(x, adj_matrix, x, W, b2)
    return out
